# baseline (speedup 1.0000x reference)
import functools

import jax
import jax.numpy as jnp
from jax.experimental import pallas as pl
from jax.experimental.pallas import tpu as pltpu


def _make_router(N, HP, H, E, B, TT, EP):
    NT = N // TT
    tiles_per_batch = NT // B

    def body(xc_ref, aq_ref, w1c_ref, w1q_ref, w2_ref, b2_ref, w_ref):
        t = pl.program_id(0)
        bidx = t // tiles_per_batch
        xc = xc_ref[...]
        hdn = jnp.dot(xc, w1c_ref[...], preferred_element_type=jnp.float32)
        abias = jnp.dot(aq_ref[...], w1q_ref[...], preferred_element_type=jnp.float32)
        rows = jax.lax.broadcasted_iota(jnp.int32, abias.shape, 0)
        brow = jnp.sum(jnp.where(rows == bidx, abias, 0.0), axis=0, keepdims=True)
        hdn = jnp.tanh(hdn + brow)
        logits = jnp.dot(hdn, w2_ref[...], preferred_element_type=jnp.float32)
        logits = logits + b2_ref[0:1, :]
        col = jax.lax.broadcasted_iota(jnp.int32, logits.shape, 1)
        valid = col < E
        logits = jnp.where(valid, logits, -jnp.inf)
        m = jnp.max(logits, axis=1, keepdims=True)
        ex = jnp.where(valid, jnp.exp(logits - m), 0.0)
        gate = ex / jnp.sum(ex, axis=1, keepdims=True)
        m1 = jnp.max(gate, axis=1, keepdims=True)
        i1 = jnp.min(jnp.where(gate == m1, col, EP + 1), axis=1, keepdims=True)
        sel1 = col == i1
        g2 = jnp.where(sel1 | ~valid, -1.0, gate)
        m2 = jnp.max(g2, axis=1, keepdims=True)
        i2 = jnp.min(jnp.where(g2 == m2, col, EP + 1), axis=1, keepdims=True)
        sel2 = col == i2
        w_ref[...] = jnp.where(sel1 | sel2, gate, 0.0)

    return body


def _make_experts(N, H, HID, E, TT, EP):
    NT = N // TT

    def body(x_ref, w_ref, wa_ref, ba_ref, wb_ref, bb_ref, out_ref, acc_ref):
        e = pl.program_id(0)
        t = pl.program_id(1)
        x = x_ref[...]
        col = jax.lax.broadcasted_iota(jnp.int32, (TT, EP), 1)
        we = jnp.sum(jnp.where(col == e, w_ref[...], 0.0), axis=1, keepdims=True)
        h1 = jnp.dot(x, wa_ref[0], preferred_element_type=jnp.float32)
        h1 = jnp.maximum(h1 + ba_ref[0], 0.0)
        y = jnp.dot(h1, wb_ref[0], preferred_element_type=jnp.float32)
        y = (y + bb_ref[0]) * we

        sl = pl.ds(t * TT, TT)

        @pl.when(e == 0)
        def _init():
            acc_ref[sl, :] = y

        @pl.when(e > 0)
        def _accum():
            acc_ref[sl, :] = acc_ref[sl, :] + y

        out_ref[...] = acc_ref[sl, :]

    return body


def kernel(token_x, aspect_q, token_score, W1, b1, W2, b2, Wa, ba, Wb, bb):
    B, M, H = token_x.shape
    E = Wa.shape[0]
    HID = Wa.shape[2]
    N = B * M
    TT = 256
    EP = 128
    NT = N // TT
    HP = H + 128

    X = token_x.reshape(N, H)
    ts = token_score.reshape(N, 1)
    Xc = jnp.concatenate([X, jnp.broadcast_to(ts, (N, 128))], axis=1)
    W1c = jnp.concatenate(
        [W1[:H], W1[2 * H][None, :], jnp.zeros((127, H), jnp.float32)], axis=0)
    aqp = jnp.zeros((8, HP), jnp.float32)
    aqp = aqp.at[:B, :H].set(aspect_q)
    aqp = aqp.at[:, H].set(1.0)
    W1q = jnp.concatenate(
        [W1[H:2 * H], b1[None, :], jnp.zeros((127, H), jnp.float32)], axis=0)
    W2p = jnp.pad(W2, ((0, 0), (0, EP - E)))
    b2p = jnp.broadcast_to(jnp.pad(b2, (0, EP - E))[None, :], (8, EP))

    w_full = pl.pallas_call(
        _make_router(N, HP, H, E, B, TT, EP),
        grid=(NT,),
        in_specs=[
            pl.BlockSpec((TT, HP), lambda t: (t, 0)),
            pl.BlockSpec((8, HP), lambda t: (0, 0)),
            pl.BlockSpec((HP, H), lambda t: (0, 0)),
            pl.BlockSpec((HP, H), lambda t: (0, 0)),
            pl.BlockSpec((H, EP), lambda t: (0, 0)),
            pl.BlockSpec((8, EP), lambda t: (0, 0)),
        ],
        out_specs=pl.BlockSpec((TT, EP), lambda t: (t, 0)),
        out_shape=jax.ShapeDtypeStruct((N, EP), jnp.float32),
    )(Xc, aqp, W1c, W1q, W2p, b2p)

    ba3 = ba.reshape(E, 1, HID)
    bb3 = bb.reshape(E, 1, H)
    out = pl.pallas_call(
        _make_experts(N, H, HID, E, TT, EP),
        grid=(E, NT),
        in_specs=[
            pl.BlockSpec((TT, H), lambda e, t: (t, 0)),
            pl.BlockSpec((TT, EP), lambda e, t: (t, 0)),
            pl.BlockSpec((1, H, HID), lambda e, t: (e, 0, 0)),
            pl.BlockSpec((1, 1, HID), lambda e, t: (e, 0, 0)),
            pl.BlockSpec((1, HID, H), lambda e, t: (e, 0, 0)),
            pl.BlockSpec((1, 1, H), lambda e, t: (e, 0, 0)),
        ],
        out_specs=pl.BlockSpec((TT, H), lambda e, t: (t, 0)),
        out_shape=jax.ShapeDtypeStruct((N, H), jnp.float32),
        scratch_shapes=[pltpu.VMEM((N, H), jnp.float32)],
    )(X, w_full, Wa, ba3, Wb, bb3)

    return out.reshape(B, M, H)
